# baseline (speedup 1.0000x reference)
import dataclasses
import functools

import jax
import jax.numpy as jnp
from jax import lax
from jax.experimental import pallas as pl
from jax.experimental.pallas import tpu as pltpu
from jax.experimental.pallas import tpu_sc as plsc

R = 8
N = 10000
E = 160000
G = 64
F = 300
FP = 384
OUT = 10
OP = 16
M = R * N
TPB = 184
BLK = TPB * 16
NBLK = 28
MP = BLK * NBLK
K = 64
EPS = 1e-5
RB = 400
NB = N // RB


def _sc_compiler_params():
    cp = pltpu.CompilerParams()
    if "needs_layout_passes" in pltpu.CompilerParams.__dataclass_fields__:
        cp = dataclasses.replace(cp, needs_layout_passes=False)
    return cp


def _spmm_sc(h, srcf, dstg, lob, hib, zrow):
    mesh = plsc.VectorSubcoreMesh(core_axis_name="c", subcore_axis_name="s")

    @functools.partial(
        pl.kernel,
        out_type=jax.ShapeDtypeStruct((MP, FP), jnp.float32),
        mesh=mesh,
        compiler_params=_sc_compiler_params(),
        scratch_types=[
            pltpu.VMEM((16,), jnp.int32),
            pltpu.VMEM((16,), jnp.int32),
            pltpu.VMEM((K,), jnp.int32),
            pltpu.VMEM((K,), jnp.int32),
            pltpu.VMEM((K,), jnp.int32),
            pltpu.VMEM((K,), jnp.int32),
            pltpu.VMEM((K, FP), jnp.float32),
            pltpu.VMEM((K, FP), jnp.float32),
            pltpu.VMEM((TPB + 8, FP), jnp.float32),
            pltpu.SemaphoreType.DMA,
            pltpu.SemaphoreType.DMA,
            pltpu.SemaphoreType.DMA,
            pltpu.SemaphoreType.DMA,
        ],
    )
    def k(h_hbm, src_hbm, dst_hbm, lo_hbm, hi_hbm, z_hbm, out_hbm,
          lo_v, hi_v, si_a, si_b, di_a, di_b, st_a, st_b, acc_v,
          sem_ga, sem_gb, sem_ia, sem_ib):
        c = lax.axis_index("c")
        s = lax.axis_index("s")
        iot = lax.iota(jnp.int32, 16)
        smask = iot == s

        def one_pass(p, _):
            b = 2 * p + c
            pltpu.sync_copy(lo_hbm.at[b], lo_v)
            pltpu.sync_copy(hi_hbm.at[b], hi_v)
            pltpu.sync_copy(z_hbm, acc_v)
            el = jnp.sum(jnp.where(smask, lo_v[...], 0))
            eh = jnp.sum(jnp.where(smask, hi_v[...], 0))
            rbase = b * BLK + s * TPB
            nch = (eh - el) // K
            npair = nch >> 1

            def idx_start(j, si, di):
                base = pl.multiple_of(el + j * K, 8)
                pltpu.async_copy(src_hbm.at[pl.ds(base, K)], si, sem(si))
                pltpu.async_copy(dst_hbm.at[pl.ds(base, K)], di, sem(si))

            def sem(ref):
                return sem_ia if ref is si_a else sem_ib

            def idx_wait(j, si, di):
                base = pl.multiple_of(el + j * K, 8)
                pltpu.make_async_copy(
                    src_hbm.at[pl.ds(base, K)], si, sem(si)).wait()
                pltpu.make_async_copy(
                    dst_hbm.at[pl.ds(base, K)], di, sem(si)).wait()

            def accumulate(st, di):
                def acc_group(g, _):
                    dvec = di[pl.ds(g * 16, 16)]
                    for l in range(16):
                        dl = dvec[l]
                        e = g * 16 + l
                        for jf in range(FP // 16):
                            fsl = pl.ds(jf * 16, 16)
                            plsc.addupdate(acc_v.at[dl, fsl], st[e, fsl])
                    return 0

                lax.fori_loop(0, K // 16, acc_group, 0)

            @pl.when(nch > 0)
            def _():
                idx_start(0, si_a, di_a)
                idx_wait(0, si_a, di_a)
                pltpu.async_copy(h_hbm.at[si_a], st_a, sem_ga)
                idx_start(1, si_b, di_b)

                def pair(q, _):
                    ja = 2 * q
                    pltpu.make_async_copy(h_hbm.at[si_a], st_a, sem_ga).wait()
                    idx_wait(ja + 1, si_b, di_b)
                    pltpu.async_copy(h_hbm.at[si_b], st_b, sem_gb)
                    accumulate(st_a, di_a)
                    idx_start(ja + 2, si_a, di_a)
                    pltpu.make_async_copy(h_hbm.at[si_b], st_b, sem_gb).wait()
                    idx_wait(ja + 2, si_a, di_a)
                    pltpu.async_copy(h_hbm.at[si_a], st_a, sem_ga)
                    accumulate(st_b, di_b)
                    idx_start(ja + 3, si_b, di_b)
                    return 0

                lax.fori_loop(0, npair, pair, 0)
                pltpu.make_async_copy(h_hbm.at[si_a], st_a, sem_ga).wait()
                idx_wait(2 * npair + 1, si_b, di_b)

            pltpu.sync_copy(acc_v.at[pl.ds(0, TPB)],
                            out_hbm.at[pl.ds(rbase, TPB)])
            return 0

        lax.fori_loop(0, NBLK // 2, one_pass, 0)

    return k(h, srcf, dstg, lob, hib, zrow)


def _h0_tc(x_pad, dropf):
    def body(x_ref, d_ref, o_ref):
        o_ref[...] = jnp.where(d_ref[...] > 0.5, 0.0, x_ref[...])

    return pl.pallas_call(
        body,
        grid=(M // RB,),
        in_specs=[pl.BlockSpec((RB, FP), lambda f: (f % NB, 0)),
                  pl.BlockSpec((RB, 1), lambda f: (f, 0))],
        out_specs=pl.BlockSpec((RB, FP), lambda f: (f, 0)),
        out_shape=jax.ShapeDtypeStruct((M, FP), jnp.float32),
    )(x_pad, dropf)


def _stage_a(h, msg, w, b):
    def body(h_ref, m_ref, w_ref, b_ref, z_ref, st_ref):
        f = pl.program_id(0)
        t = h_ref[...] + m_ref[...]
        z = jnp.dot(t, w_ref[...], preferred_element_type=jnp.float32) + b_ref[...]
        z_ref[...] = z

        @pl.when(f == 0)
        def _():
            st_ref[...] = jnp.zeros_like(st_ref)

        st_ref[0:1, :] += jnp.sum(z, axis=0, keepdims=True)
        st_ref[1:2, :] += jnp.sum(z * z, axis=0, keepdims=True)

    return pl.pallas_call(
        body,
        grid=(M // RB,),
        in_specs=[pl.BlockSpec((RB, FP), lambda f: (f, 0)),
                  pl.BlockSpec((RB, FP), lambda f: (f, 0)),
                  pl.BlockSpec((FP, FP), lambda f: (0, 0)),
                  pl.BlockSpec((1, FP), lambda f: (0, 0))],
        out_specs=[pl.BlockSpec((RB, FP), lambda f: (f, 0)),
                   pl.BlockSpec((8, FP), lambda f: (0, 0))],
        out_shape=[jax.ShapeDtypeStruct((M, FP), jnp.float32),
                   jax.ShapeDtypeStruct((8, FP), jnp.float32)],
    )(h, msg, w, b)


def _bn_affine(s_ref, g_ref, bt_ref):
    mean = s_ref[0:1, :] * (1.0 / M)
    var = s_ref[1:2, :] * (1.0 / M) - mean * mean
    a = g_ref[...] * lax.rsqrt(var + EPS)
    bb = bt_ref[...] - mean * a
    return a, bb


def _stage_b(z1, st1, g1, bt1, w, b):
    def body(z_ref, s_ref, g_ref, bt_ref, w_ref, b_ref, o_ref, st_ref):
        f = pl.program_id(0)
        a, bb = _bn_affine(s_ref, g_ref, bt_ref)
        t = jnp.maximum(z_ref[...] * a + bb, 0.0)
        z = jnp.dot(t, w_ref[...], preferred_element_type=jnp.float32) + b_ref[...]
        o_ref[...] = z

        @pl.when(f == 0)
        def _():
            st_ref[...] = jnp.zeros_like(st_ref)

        st_ref[0:1, :] += jnp.sum(z, axis=0, keepdims=True)
        st_ref[1:2, :] += jnp.sum(z * z, axis=0, keepdims=True)

    return pl.pallas_call(
        body,
        grid=(M // RB,),
        in_specs=[pl.BlockSpec((RB, FP), lambda f: (f, 0)),
                  pl.BlockSpec((8, FP), lambda f: (0, 0)),
                  pl.BlockSpec((1, FP), lambda f: (0, 0)),
                  pl.BlockSpec((1, FP), lambda f: (0, 0)),
                  pl.BlockSpec((FP, FP), lambda f: (0, 0)),
                  pl.BlockSpec((1, FP), lambda f: (0, 0))],
        out_specs=[pl.BlockSpec((RB, FP), lambda f: (f, 0)),
                   pl.BlockSpec((8, FP), lambda f: (0, 0))],
        out_shape=[jax.ShapeDtypeStruct((M, FP), jnp.float32),
                   jax.ShapeDtypeStruct((8, FP), jnp.float32)],
    )(z1, st1, g1, bt1, w, b)


def _stage_c(z2, st2, g, bt):
    def body(z_ref, s_ref, g_ref, bt_ref, h_ref, sum_ref):
        r = pl.program_id(1)
        a, bb = _bn_affine(s_ref, g_ref, bt_ref)
        val = jnp.maximum(z_ref[...] * a + bb, 0.0)
        h_ref[...] = val

        @pl.when(r == 0)
        def _():
            sum_ref[...] = jnp.zeros_like(sum_ref)

        sum_ref[...] += val

    return pl.pallas_call(
        body,
        grid=(NB, R),
        in_specs=[pl.BlockSpec((RB, FP), lambda n, r: (r * NB + n, 0)),
                  pl.BlockSpec((8, FP), lambda n, r: (0, 0)),
                  pl.BlockSpec((1, FP), lambda n, r: (0, 0)),
                  pl.BlockSpec((1, FP), lambda n, r: (0, 0))],
        out_specs=[pl.BlockSpec((RB, FP), lambda n, r: (r * NB + n, 0)),
                   pl.BlockSpec((RB, FP), lambda n, r: (n, 0))],
        out_shape=[jax.ShapeDtypeStruct((M, FP), jnp.float32),
                   jax.ShapeDtypeStruct((N, FP), jnp.float32)],
    )(z2, st2, g, bt)


def _tail(x_pad, drop_nr, s1, s2, s3, s4, w0, w1, w2, w3, w4, batch_row):
    def body(x_ref, d_ref, s1_ref, s2_ref, s3_ref, s4_ref,
             w0_ref, w1_ref, w2_ref, w3_ref, w4_ref, bt_ref,
             o_ref, c_ref):
        nb = pl.program_id(0)
        dc = jnp.sum(d_ref[...], axis=1, keepdims=True)
        m0 = x_ref[...] * ((R - dc) * (1.0 / R))
        p = jnp.dot(m0, w0_ref[...], preferred_element_type=jnp.float32)
        for s_ref, w_ref in ((s1_ref, w1_ref), (s2_ref, w2_ref),
                             (s3_ref, w3_ref), (s4_ref, w4_ref)):
            p += jnp.dot(s_ref[...] * (1.0 / R), w_ref[...],
                         preferred_element_type=jnp.float32)
        ii = lax.broadcasted_iota(jnp.int32, (G, RB), 0)
        oh = (ii == bt_ref[...].reshape(1, RB)).astype(jnp.float32)

        @pl.when(nb == 0)
        def _():
            o_ref[...] = jnp.zeros_like(o_ref)
            c_ref[...] = jnp.zeros_like(c_ref)

        o_ref[...] += jnp.dot(oh, p, preferred_element_type=jnp.float32)
        c_ref[...] += jnp.sum(oh, axis=1, keepdims=True)

    full = lambda n: (0, 0)
    blk = lambda n: (n, 0)
    return pl.pallas_call(
        body,
        grid=(NB,),
        in_specs=[pl.BlockSpec((RB, FP), blk),
                  pl.BlockSpec((RB, R), blk),
                  pl.BlockSpec((RB, FP), blk),
                  pl.BlockSpec((RB, FP), blk),
                  pl.BlockSpec((RB, FP), blk),
                  pl.BlockSpec((RB, FP), blk),
                  pl.BlockSpec((FP, OP), full),
                  pl.BlockSpec((FP, OP), full),
                  pl.BlockSpec((FP, OP), full),
                  pl.BlockSpec((FP, OP), full),
                  pl.BlockSpec((FP, OP), full),
                  pl.BlockSpec((1, 1, RB), lambda n: (n, 0, 0))],
        out_specs=[pl.BlockSpec((G, OP), full),
                   pl.BlockSpec((G, 1), full)],
        out_shape=[jax.ShapeDtypeStruct((G, OP), jnp.float32),
                   jax.ShapeDtypeStruct((G, 1), jnp.float32)],
    )(x_pad, drop_nr, s1, s2, s3, s4, w0, w1, w2, w3, w4, batch_row)


def _pad_mat(w, rows, cols):
    return jnp.pad(w, ((0, rows - w.shape[0]), (0, cols - w.shape[1])))


def _pad_vec(v):
    return jnp.pad(v, (0, FP - v.shape[0])).reshape(1, FP)


def kernel(x, edge_index, batch, drop, params):
    src = edge_index[0].astype(jnp.int32)
    dst = edge_index[1].astype(jnp.int32)
    offset = jnp.max(edge_index).astype(jnp.int32) + 1
    order = jnp.argsort(dst)
    src_s = src[order]
    dst_s = dst[order]
    roff = jnp.arange(R, dtype=jnp.int32) * offset
    srcf = (src_s[None, :] + roff[:, None]).reshape(-1)
    dstf = (dst_s[None, :] + roff[:, None]).reshape(-1)
    NRNG = NBLK * 16
    PADK = 2 * K
    EP = R * E + NRNG * PADK + 2 * K
    etot = R * E
    bnd = jnp.searchsorted(
        dstf, jnp.arange(NRNG + 1, dtype=jnp.int32) * TPB).astype(jnp.int32)
    cnt = bnd[1:] - bnd[:-1]
    pc = ((cnt + PADK - 1) // PADK) * PADK
    starts = jnp.concatenate([jnp.zeros((1,), jnp.int32),
                              jnp.cumsum(pc).astype(jnp.int32)])
    ii = jnp.arange(EP, dtype=jnp.int32)
    rid = jnp.clip(jnp.searchsorted(starts, ii, side="right") - 1, 0, NRNG - 1)
    off = ii - starts[rid]
    e_i = jnp.clip(bnd[rid] + off, 0, etot - 1)
    valid = off < cnt[rid]
    srcp = jnp.where(valid, srcf[e_i], 0)
    dstp = jnp.where(valid, dstf[e_i] - rid * TPB, TPB).astype(jnp.int32)
    lob = starts[:NRNG].reshape(NBLK, 16)
    hib = (starts[:NRNG] + pc).reshape(NBLK, 16)
    zrow = jnp.zeros((TPB + 8, FP), jnp.float32)

    x_pad = jnp.pad(x, ((0, 0), (0, FP - F)))
    dropf = drop.astype(jnp.float32).reshape(M, 1)
    drop_nr = drop.astype(jnp.float32).T
    batch_row = batch.astype(jnp.int32).reshape(NB, 1, RB)

    convs = params["convs"]
    bns = params["bns"]
    fcs = params["fcs"]
    w1p = [_pad_mat(c["W1"], FP, FP) for c in convs]
    b1p = [_pad_vec(c["b1"]) for c in convs]
    g1p = [_pad_vec(c["g1"]) for c in convs]
    bt1p = [_pad_vec(c["bt1"]) for c in convs]
    w2p = [_pad_mat(c["W2"], FP, FP) for c in convs]
    b2p = [_pad_vec(c["b2"]) for c in convs]
    gp = [_pad_vec(b["g"]) for b in bns]
    bp = [_pad_vec(b["b"]) for b in bns]
    wfp = [_pad_mat(fc["W"], FP, OP) for fc in fcs]

    h = _h0_tc(x_pad, dropf)
    sums = []
    for i in range(4):
        msg = _spmm_sc(h, srcp, dstp, lob, hib, zrow)
        z1, st1 = _stage_a(h, msg, w1p[i], b1p[i])
        z2, st2 = _stage_b(z1, st1, g1p[i], bt1p[i], w2p[i], b2p[i])
        h, s_i = _stage_c(z2, st2, gp[i], bp[i])
        sums.append(s_i)

    o_acc, cnt = _tail(x_pad, drop_nr, *sums, *wfp, batch_row)
    bsum = sum(fc["b"] for fc in fcs)
    return o_acc[:, :OUT] / jnp.maximum(cnt, 1.0) + bsum[None, :]


# speedup vs baseline: 6.6290x; 6.6290x over previous
import dataclasses
import functools

import jax
import jax.numpy as jnp
from jax import lax
from jax.experimental import pallas as pl
from jax.experimental.pallas import tpu as pltpu
from jax.experimental.pallas import tpu_sc as plsc

R = 8
N = 10000
E = 160000
G = 64
F = 300
FP = 384
OUT = 10
OP = 16
M = R * N
TPB = 184
BLK = TPB * 16
NBLK = 28
MP = BLK * NBLK
K = 64
EPS = 1e-5
RB = 400
NB = N // RB


def _sc_compiler_params():
    cp = pltpu.CompilerParams()
    if "needs_layout_passes" in pltpu.CompilerParams.__dataclass_fields__:
        cp = dataclasses.replace(cp, needs_layout_passes=False)
    return cp


def _spmm_sc(h, srcf, dstg, lob, hib, zrow):
    mesh = plsc.VectorSubcoreMesh(core_axis_name="c", subcore_axis_name="s")

    @functools.partial(
        pl.kernel,
        out_type=jax.ShapeDtypeStruct((MP, FP), jnp.float32),
        mesh=mesh,
        compiler_params=_sc_compiler_params(),
        scratch_types=[
            pltpu.VMEM((16,), jnp.int32),
            pltpu.VMEM((16,), jnp.int32),
            pltpu.VMEM((K,), jnp.int32),
            pltpu.VMEM((K,), jnp.int32),
            pltpu.VMEM((K,), jnp.int32),
            pltpu.VMEM((K,), jnp.int32),
            pltpu.VMEM((K, FP), jnp.float32),
            pltpu.VMEM((K, FP), jnp.float32),
            pltpu.VMEM((TPB + 8, FP), jnp.float32),
            pltpu.SemaphoreType.DMA,
            pltpu.SemaphoreType.DMA,
            pltpu.SemaphoreType.DMA,
            pltpu.SemaphoreType.DMA,
        ],
    )
    def k(h_hbm, src_hbm, dst_hbm, lo_hbm, hi_hbm, z_hbm, out_hbm,
          lo_v, hi_v, si_a, si_b, di_a, di_b, st_a, st_b, acc_v,
          sem_ga, sem_gb, sem_ia, sem_ib):
        c = lax.axis_index("c")
        s = lax.axis_index("s")
        iot = lax.iota(jnp.int32, 16)
        smask = iot == s

        def one_pass(p, _):
            b = 2 * p + c
            pltpu.sync_copy(lo_hbm.at[b], lo_v)
            pltpu.sync_copy(hi_hbm.at[b], hi_v)
            pltpu.sync_copy(z_hbm, acc_v)
            el = jnp.sum(jnp.where(smask, lo_v[...], 0))
            eh = jnp.sum(jnp.where(smask, hi_v[...], 0))
            rbase = b * BLK + s * TPB
            el8 = el & (-8)
            nch = ((eh - el8) + (K - 1)) // K
            npair = (nch + 1) >> 1

            def idx_start(j, si, di):
                base = pl.multiple_of(el8 + j * K, 8)
                pltpu.async_copy(src_hbm.at[pl.ds(base, K)], si, sem(si))
                pltpu.async_copy(dst_hbm.at[pl.ds(base, K)], di, sem(si))

            def sem(ref):
                return sem_ia if ref is si_a else sem_ib

            def idx_wait(j, si, di):
                base = pl.multiple_of(el8 + j * K, 8)
                pltpu.make_async_copy(
                    src_hbm.at[pl.ds(base, K)], si, sem(si)).wait()
                pltpu.make_async_copy(
                    dst_hbm.at[pl.ds(base, K)], di, sem(si)).wait()

            def mask(j, si, di):
                base = el8 + j * K
                for v in range(K // 16):
                    lane = base + v * 16 + iot
                    valid = (lane >= el) & (lane < eh)
                    sl = pl.ds(v * 16, 16)
                    si[sl] = jnp.where(valid, si[sl], 0)
                    di[sl] = jnp.where(valid, di[sl] - rbase, TPB)

            def accumulate(st, di):
                def acc_group(g, _):
                    dvec = di[pl.ds(g * 16, 16)]
                    for l in range(16):
                        dl = dvec[l]
                        e = g * 16 + l
                        for jf in range(FP // 16):
                            fsl = pl.ds(jf * 16, 16)
                            plsc.addupdate(acc_v.at[dl, fsl], st[e, fsl])
                    return 0

                lax.fori_loop(0, K // 16, acc_group, 0)

            @pl.when(nch > 0)
            def _():
                idx_start(0, si_a, di_a)
                idx_wait(0, si_a, di_a)
                mask(0, si_a, di_a)
                pltpu.async_copy(h_hbm.at[si_a], st_a, sem_ga)
                idx_start(1, si_b, di_b)

                def pair(q, _):
                    ja = 2 * q
                    pltpu.make_async_copy(h_hbm.at[si_a], st_a, sem_ga).wait()
                    idx_wait(ja + 1, si_b, di_b)
                    mask(ja + 1, si_b, di_b)
                    pltpu.async_copy(h_hbm.at[si_b], st_b, sem_gb)
                    accumulate(st_a, di_a)
                    idx_start(ja + 2, si_a, di_a)
                    pltpu.make_async_copy(h_hbm.at[si_b], st_b, sem_gb).wait()
                    idx_wait(ja + 2, si_a, di_a)
                    mask(ja + 2, si_a, di_a)
                    pltpu.async_copy(h_hbm.at[si_a], st_a, sem_ga)
                    accumulate(st_b, di_b)
                    idx_start(ja + 3, si_b, di_b)
                    return 0

                lax.fori_loop(0, npair, pair, 0)
                pltpu.make_async_copy(h_hbm.at[si_a], st_a, sem_ga).wait()
                idx_wait(2 * npair + 1, si_b, di_b)

            pltpu.sync_copy(acc_v.at[pl.ds(0, TPB)],
                            out_hbm.at[pl.ds(rbase, TPB)])
            return 0

        lax.fori_loop(0, NBLK // 2, one_pass, 0)

    return k(h, srcf, dstg, lob, hib, zrow)


def _h0_tc(x_pad, dropf):
    def body(x_ref, d_ref, o_ref):
        o_ref[...] = jnp.where(d_ref[...] > 0.5, 0.0, x_ref[...])

    return pl.pallas_call(
        body,
        grid=(M // RB,),
        in_specs=[pl.BlockSpec((RB, FP), lambda f: (f % NB, 0)),
                  pl.BlockSpec((RB, 1), lambda f: (f, 0))],
        out_specs=pl.BlockSpec((RB, FP), lambda f: (f, 0)),
        out_shape=jax.ShapeDtypeStruct((M, FP), jnp.float32),
    )(x_pad, dropf)


def _stage_a(h, msg, w, b):
    def body(h_ref, m_ref, w_ref, b_ref, z_ref, st_ref):
        f = pl.program_id(0)
        t = h_ref[...] + m_ref[...]
        z = jnp.dot(t, w_ref[...], preferred_element_type=jnp.float32) + b_ref[...]
        z_ref[...] = z

        @pl.when(f == 0)
        def _():
            st_ref[...] = jnp.zeros_like(st_ref)

        st_ref[0:1, :] += jnp.sum(z, axis=0, keepdims=True)
        st_ref[1:2, :] += jnp.sum(z * z, axis=0, keepdims=True)

    return pl.pallas_call(
        body,
        grid=(M // RB,),
        in_specs=[pl.BlockSpec((RB, FP), lambda f: (f, 0)),
                  pl.BlockSpec((RB, FP), lambda f: (f, 0)),
                  pl.BlockSpec((FP, FP), lambda f: (0, 0)),
                  pl.BlockSpec((1, FP), lambda f: (0, 0))],
        out_specs=[pl.BlockSpec((RB, FP), lambda f: (f, 0)),
                   pl.BlockSpec((8, FP), lambda f: (0, 0))],
        out_shape=[jax.ShapeDtypeStruct((M, FP), jnp.float32),
                   jax.ShapeDtypeStruct((8, FP), jnp.float32)],
    )(h, msg, w, b)


def _bn_affine(s_ref, g_ref, bt_ref):
    mean = s_ref[0:1, :] * (1.0 / M)
    var = s_ref[1:2, :] * (1.0 / M) - mean * mean
    a = g_ref[...] * lax.rsqrt(var + EPS)
    bb = bt_ref[...] - mean * a
    return a, bb


def _stage_b(z1, st1, g1, bt1, w, b):
    def body(z_ref, s_ref, g_ref, bt_ref, w_ref, b_ref, o_ref, st_ref):
        f = pl.program_id(0)
        a, bb = _bn_affine(s_ref, g_ref, bt_ref)
        t = jnp.maximum(z_ref[...] * a + bb, 0.0)
        z = jnp.dot(t, w_ref[...], preferred_element_type=jnp.float32) + b_ref[...]
        o_ref[...] = z

        @pl.when(f == 0)
        def _():
            st_ref[...] = jnp.zeros_like(st_ref)

        st_ref[0:1, :] += jnp.sum(z, axis=0, keepdims=True)
        st_ref[1:2, :] += jnp.sum(z * z, axis=0, keepdims=True)

    return pl.pallas_call(
        body,
        grid=(M // RB,),
        in_specs=[pl.BlockSpec((RB, FP), lambda f: (f, 0)),
                  pl.BlockSpec((8, FP), lambda f: (0, 0)),
                  pl.BlockSpec((1, FP), lambda f: (0, 0)),
                  pl.BlockSpec((1, FP), lambda f: (0, 0)),
                  pl.BlockSpec((FP, FP), lambda f: (0, 0)),
                  pl.BlockSpec((1, FP), lambda f: (0, 0))],
        out_specs=[pl.BlockSpec((RB, FP), lambda f: (f, 0)),
                   pl.BlockSpec((8, FP), lambda f: (0, 0))],
        out_shape=[jax.ShapeDtypeStruct((M, FP), jnp.float32),
                   jax.ShapeDtypeStruct((8, FP), jnp.float32)],
    )(z1, st1, g1, bt1, w, b)


def _stage_c(z2, st2, g, bt):
    def body(z_ref, s_ref, g_ref, bt_ref, h_ref, sum_ref):
        r = pl.program_id(1)
        a, bb = _bn_affine(s_ref, g_ref, bt_ref)
        val = jnp.maximum(z_ref[...] * a + bb, 0.0)
        h_ref[...] = val

        @pl.when(r == 0)
        def _():
            sum_ref[...] = jnp.zeros_like(sum_ref)

        sum_ref[...] += val

    return pl.pallas_call(
        body,
        grid=(NB, R),
        in_specs=[pl.BlockSpec((RB, FP), lambda n, r: (r * NB + n, 0)),
                  pl.BlockSpec((8, FP), lambda n, r: (0, 0)),
                  pl.BlockSpec((1, FP), lambda n, r: (0, 0)),
                  pl.BlockSpec((1, FP), lambda n, r: (0, 0))],
        out_specs=[pl.BlockSpec((RB, FP), lambda n, r: (r * NB + n, 0)),
                   pl.BlockSpec((RB, FP), lambda n, r: (n, 0))],
        out_shape=[jax.ShapeDtypeStruct((M, FP), jnp.float32),
                   jax.ShapeDtypeStruct((N, FP), jnp.float32)],
    )(z2, st2, g, bt)


def _tail(x_pad, drop_nr, s1, s2, s3, s4, w0, w1, w2, w3, w4, batch_row):
    def body(x_ref, d_ref, s1_ref, s2_ref, s3_ref, s4_ref,
             w0_ref, w1_ref, w2_ref, w3_ref, w4_ref, bt_ref,
             o_ref, c_ref):
        nb = pl.program_id(0)
        dc = jnp.sum(d_ref[...], axis=1, keepdims=True)
        m0 = x_ref[...] * ((R - dc) * (1.0 / R))
        p = jnp.dot(m0, w0_ref[...], preferred_element_type=jnp.float32)
        for s_ref, w_ref in ((s1_ref, w1_ref), (s2_ref, w2_ref),
                             (s3_ref, w3_ref), (s4_ref, w4_ref)):
            p += jnp.dot(s_ref[...] * (1.0 / R), w_ref[...],
                         preferred_element_type=jnp.float32)
        ii = lax.broadcasted_iota(jnp.int32, (G, RB), 0)
        oh = (ii == bt_ref[...].reshape(1, RB)).astype(jnp.float32)

        @pl.when(nb == 0)
        def _():
            o_ref[...] = jnp.zeros_like(o_ref)
            c_ref[...] = jnp.zeros_like(c_ref)

        o_ref[...] += jnp.dot(oh, p, preferred_element_type=jnp.float32)
        c_ref[...] += jnp.sum(oh, axis=1, keepdims=True)

    full = lambda n: (0, 0)
    blk = lambda n: (n, 0)
    return pl.pallas_call(
        body,
        grid=(NB,),
        in_specs=[pl.BlockSpec((RB, FP), blk),
                  pl.BlockSpec((RB, R), blk),
                  pl.BlockSpec((RB, FP), blk),
                  pl.BlockSpec((RB, FP), blk),
                  pl.BlockSpec((RB, FP), blk),
                  pl.BlockSpec((RB, FP), blk),
                  pl.BlockSpec((FP, OP), full),
                  pl.BlockSpec((FP, OP), full),
                  pl.BlockSpec((FP, OP), full),
                  pl.BlockSpec((FP, OP), full),
                  pl.BlockSpec((FP, OP), full),
                  pl.BlockSpec((1, 1, RB), lambda n: (n, 0, 0))],
        out_specs=[pl.BlockSpec((G, OP), full),
                   pl.BlockSpec((G, 1), full)],
        out_shape=[jax.ShapeDtypeStruct((G, OP), jnp.float32),
                   jax.ShapeDtypeStruct((G, 1), jnp.float32)],
    )(x_pad, drop_nr, s1, s2, s3, s4, w0, w1, w2, w3, w4, batch_row)


def _pad_mat(w, rows, cols):
    return jnp.pad(w, ((0, rows - w.shape[0]), (0, cols - w.shape[1])))


def _pad_vec(v):
    return jnp.pad(v, (0, FP - v.shape[0])).reshape(1, FP)


def kernel(x, edge_index, batch, drop, params):
    src = edge_index[0].astype(jnp.int32)
    dst = edge_index[1].astype(jnp.int32)
    offset = jnp.max(edge_index).astype(jnp.int32) + 1
    order = jnp.argsort(dst)
    src_s = src[order]
    dst_s = dst[order]
    roff = jnp.arange(R, dtype=jnp.int32) * offset
    srcf = (src_s[None, :] + roff[:, None]).reshape(-1)
    dstf = (dst_s[None, :] + roff[:, None]).reshape(-1)
    NRNG = NBLK * 16
    srcp = jnp.concatenate([srcf, jnp.zeros((4 * K,), jnp.int32)])
    dstp = jnp.concatenate([dstf, jnp.zeros((4 * K,), jnp.int32)])
    bnd = jnp.searchsorted(
        dstf, jnp.arange(NRNG + 1, dtype=jnp.int32) * TPB).astype(jnp.int32)
    lob = bnd[:NRNG].reshape(NBLK, 16)
    hib = bnd[1:NRNG + 1].reshape(NBLK, 16)
    zrow = jnp.zeros((TPB + 8, FP), jnp.float32)

    x_pad = jnp.pad(x, ((0, 0), (0, FP - F)))
    dropf = drop.astype(jnp.float32).reshape(M, 1)
    drop_nr = drop.astype(jnp.float32).T
    batch_row = batch.astype(jnp.int32).reshape(NB, 1, RB)

    convs = params["convs"]
    bns = params["bns"]
    fcs = params["fcs"]
    w1p = [_pad_mat(c["W1"], FP, FP) for c in convs]
    b1p = [_pad_vec(c["b1"]) for c in convs]
    g1p = [_pad_vec(c["g1"]) for c in convs]
    bt1p = [_pad_vec(c["bt1"]) for c in convs]
    w2p = [_pad_mat(c["W2"], FP, FP) for c in convs]
    b2p = [_pad_vec(c["b2"]) for c in convs]
    gp = [_pad_vec(b["g"]) for b in bns]
    bp = [_pad_vec(b["b"]) for b in bns]
    wfp = [_pad_mat(fc["W"], FP, OP) for fc in fcs]

    h = _h0_tc(x_pad, dropf)
    sums = []
    for i in range(4):
        msg = _spmm_sc(h, srcp, dstp, lob, hib, zrow)
        z1, st1 = _stage_a(h, msg, w1p[i], b1p[i])
        z2, st2 = _stage_b(z1, st1, g1p[i], bt1p[i], w2p[i], b2p[i])
        h, s_i = _stage_c(z2, st2, gp[i], bp[i])
        sums.append(s_i)

    o_acc, cnt = _tail(x_pad, drop_nr, *sums, *wfp, batch_row)
    bsum = sum(fc["b"] for fc in fcs)
    return o_acc[:, :OUT] / jnp.maximum(cnt, 1.0) + bsum[None, :]
